# Initial kernel scaffold; baseline (speedup 1.0000x reference)
import jax
import jax.numpy as jnp
from jax.experimental import pallas as pl


def kernel(user_id, item_id, user_feat, item_feat, edge_feature, num_sampling, user2item, item2user, W_proj_u, b_proj_u, W_proj_i, b_proj_i, Wu_self0, Wu_agg0, Wu_self1, Wu_agg1, Wi_self0, Wi_agg0, Wi_self1, Wi_agg1, W_lin, b_lin):
    raise NotImplementedError("write your pallas kernel here")


# speedup vs baseline: 2.9816x; 2.9816x over previous
import functools

import jax
import jax.numpy as jnp
from jax import lax
from jax.experimental import pallas as pl
from jax.experimental.pallas import tpu as pltpu
from jax.experimental.pallas import tpu_sc as plsc

F32 = jnp.float32
I32 = jnp.int32

B = 1024
D = 256
HOP = 10
N1 = B * HOP
NC, NS = 2, 16
NW = NC * NS
IDS_W = B // NW
S1_W = IDS_W * HOP
S2_W = S1_W * HOP
CH = 80
NCH = S2_W // CH
NXCH = S1_W // CH
HG = S1_W // 2
ACC = NS * HG
INV_HOP = 0.1


def _proj_body(x_ref, w_ref, b_ref, o_ref):
    y = jnp.dot(x_ref[...], w_ref[...], preferred_element_type=F32)
    o_ref[...] = jax.nn.sigmoid(y + b_ref[...])


def _project(feat, w, b):
    n = feat.shape[0]
    blk = 1000
    return pl.pallas_call(
        _proj_body,
        grid=(n // blk,),
        in_specs=[
            pl.BlockSpec((blk, D), lambda i: (i, 0)),
            pl.BlockSpec((D, D), lambda i: (0, 0)),
            pl.BlockSpec((1, D), lambda i: (0, 0)),
        ],
        out_specs=pl.BlockSpec((blk, D), lambda i: (i, 0)),
        out_shape=jax.ShapeDtypeStruct((n, D), F32),
    )(feat, w, b.reshape(1, D))


def _pos16(off):
    return lax.iota(I32, 16) + off


def _div10(pos):
    return (pos.astype(F32) * F32(0.1)).astype(I32)


def _sc_body(uid, iid, up, ip, u2i, i2u, x0_o, x1_o, m2_o,
             ids_v, s1_v, s2_v, idx_v, rows_v, acc_v):
    c = lax.axis_index("c")
    s = lax.axis_index("s")
    wid = c * NS + s

    sides = ((uid, up, ip, u2i, i2u), (iid, ip, up, i2u, u2i))
    for side, (seed, projA, projB, adjA, adjB) in enumerate(sides):
        pltpu.sync_copy(seed.at[pl.ds(wid * IDS_W, IDS_W)], ids_v)

        for i in range(S1_W // 16):
            pos = _pos16(i * 16)
            r = _div10(pos)
            rb = min((i * 16) // HOP, IDS_W - 16)
            v = ids_v[pl.ds(rb, 16)][r - rb]
            idx_v[pl.ds(i * 16, 16)] = v * 16 + (pos - r * HOP)

        def _g1(i, _):
            pltpu.sync_copy(adjA.at[idx_v.at[pl.ds(i * CH, CH)]],
                            s1_v.at[pl.ds(i * CH, CH)])
            return 0
        lax.fori_loop(0, NXCH, _g1, 0)

        for i in range(S2_W // 16):
            pos = _pos16(i * 16)
            r = _div10(pos)
            rb = min((i * 16) // HOP, S1_W - 16)
            v = s1_v[pl.ds(rb, 16)][r - rb]
            idx_v[pl.ds(i * 16, 16)] = v * 16 + (pos - r * HOP)

        def _g2(i, _):
            pltpu.sync_copy(adjB.at[idx_v.at[pl.ds(i * 128, 128)]],
                            s2_v.at[pl.ds(i * 128, 128)])
            return 0
        lax.fori_loop(0, S2_W // 128, _g2, 0)

        pltpu.sync_copy(projA.at[ids_v], rows_v.at[pl.ds(0, IDS_W)])
        pltpu.sync_copy(rows_v.at[pl.ds(0, IDS_W)],
                        x0_o.at[side].at[pl.ds(wid * IDS_W, IDS_W)])

        def _x1(i, _):
            pltpu.sync_copy(projB.at[s1_v.at[pl.ds(i * CH, CH)]], rows_v)
            pltpu.sync_copy(
                rows_v, x1_o.at[side].at[pl.ds(wid * S1_W + i * CH, CH)])
            return 0
        lax.fori_loop(0, NXCH, _x1, 0)

        def _m2(jg, _):
            pltpu.sync_copy(projA.at[s2_v.at[pl.ds(jg * CH, CH)]], rows_v)

            def _grp(g, _):
                row0 = g * HOP

                def _col(cq, _):
                    cs = pl.ds(cq * 16, 16)
                    x = rows_v[row0, cs]
                    for t in range(1, HOP):
                        x = x + rows_v[row0 + t, cs]
                    acc_v[jg * (CH // HOP) + g, cs] = x
                    return 0
                return lax.fori_loop(0, D // 16, _col, 0)
            lax.fori_loop(0, CH // HOP, _grp, 0)
            return 0
        lax.fori_loop(0, NCH, _m2, 0)

        pltpu.sync_copy(acc_v, m2_o.at[side].at[pl.ds(wid * S1_W, S1_W)])


def _sc_stage(user_id, item_id, up, ip, u2i, i2u):
    mesh = plsc.VectorSubcoreMesh(core_axis_name="c", subcore_axis_name="s")
    fn = pl.kernel(
        _sc_body,
        out_type=[
            jax.ShapeDtypeStruct((2, B, D), F32),
            jax.ShapeDtypeStruct((2, N1, D), F32),
            jax.ShapeDtypeStruct((2, N1, D), F32),
        ],
        mesh=mesh,
        scratch_types=[
            pltpu.VMEM((IDS_W,), I32),
            pltpu.VMEM((S1_W,), I32),
            pltpu.VMEM((S2_W,), I32),
            pltpu.VMEM((S2_W,), I32),
            pltpu.VMEM((CH, D), F32),
            pltpu.VMEM((S1_W, D), F32),
        ],
    )
    return fn(user_id, item_id, up, ip, u2i, i2u)


def _l0_body(x1_ref, m2_ref, ws_ref, wa_ref, h1s_ref, xs1_ref):
    x1 = x1_ref[0]
    m2 = m2_ref[0] * INV_HOP
    h1 = jnp.dot(x1, ws_ref[0], preferred_element_type=F32)
    h1 = h1 + jnp.dot(m2, wa_ref[0], preferred_element_type=F32)
    h1 = jnp.maximum(h1, 0.0)
    rows = h1.shape[0] // HOP
    h1s_ref[0] = h1.reshape(rows, HOP, D).sum(axis=1)
    xs1_ref[0] = x1.reshape(rows, HOP, D).sum(axis=1)


def _layer0(x1s, m2s, ws0, wa0):
    blk = 1280
    nblk = N1 // blk
    return pl.pallas_call(
        _l0_body,
        grid=(2, nblk),
        in_specs=[
            pl.BlockSpec((1, blk, D), lambda g, i: (g, i, 0)),
            pl.BlockSpec((1, blk, D), lambda g, i: (g, i, 0)),
            pl.BlockSpec((1, D, D), lambda g, i: (g, 0, 0)),
            pl.BlockSpec((1, D, D), lambda g, i: (g, 0, 0)),
        ],
        out_specs=[
            pl.BlockSpec((1, blk // HOP, D), lambda g, i: (g, i, 0)),
            pl.BlockSpec((1, blk // HOP, D), lambda g, i: (g, i, 0)),
        ],
        out_shape=[
            jax.ShapeDtypeStruct((2, B, D), F32),
            jax.ShapeDtypeStruct((2, B, D), F32),
        ],
    )(x1s, m2s, ws0, wa0)


def _head_body(x0_ref, xs1_ref, h1s_ref, ws0_ref, wa0_ref, ws1_ref, wa1_ref,
               wl_ref, bl_ref, o_ref):
    outs = []
    for g in range(2):
        h0 = jnp.dot(x0_ref[g], ws0_ref[g], preferred_element_type=F32)
        h0 = h0 + jnp.dot(xs1_ref[g] * INV_HOP, wa0_ref[g],
                          preferred_element_type=F32)
        h0 = jnp.maximum(h0, 0.0)
        hid = jnp.dot(h0, ws1_ref[g], preferred_element_type=F32)
        hid = hid + jnp.dot(h1s_ref[g] * INV_HOP, wa1_ref[g],
                            preferred_element_type=F32)
        outs.append(hid)
    prod = outs[0] * outs[1]
    o_ref[...] = jnp.sum(prod * wl_ref[...], axis=-1,
                         keepdims=True) + bl_ref[...]


def _head(x0s, xs1, h1s, ws0, wa0, ws1, wa1, wl, bl):
    h1 = ws1.shape[-1]
    return pl.pallas_call(
        _head_body,
        out_shape=jax.ShapeDtypeStruct((B, 1), F32),
    )(x0s, xs1, h1s, ws0, wa0, ws1, wa1, wl.reshape(1, h1), bl.reshape(1, 1))


def kernel(user_id, item_id, user_feat, item_feat, edge_feature, num_sampling,
           user2item, item2user, W_proj_u, b_proj_u, W_proj_i, b_proj_i,
           Wu_self0, Wu_agg0, Wu_self1, Wu_agg1, Wi_self0, Wi_agg0, Wi_self1,
           Wi_agg1, W_lin, b_lin):
    up = _project(user_feat, W_proj_u, b_proj_u)
    ip = _project(item_feat, W_proj_i, b_proj_i)

    x0s, x1s, m2s = _sc_stage(user_id.astype(I32), item_id.astype(I32),
                              up, ip, user2item.astype(I32).reshape(-1),
                              item2user.astype(I32).reshape(-1))

    ws0 = jnp.stack([Wu_self0, Wi_self0])
    wa0 = jnp.stack([Wu_agg0, Wi_agg0])
    ws1 = jnp.stack([Wu_self1, Wi_self1])
    wa1 = jnp.stack([Wu_agg1, Wi_agg1])

    h1s, xs1 = _layer0(x1s, m2s, ws0, wa0)
    pred = _head(x0s, xs1, h1s, ws0, wa0, ws1, wa1, W_lin, b_lin)
    return (pred, num_sampling)


# speedup vs baseline: 3.8856x; 1.3032x over previous
import functools

import jax
import jax.numpy as jnp
from jax import lax
from jax.experimental import pallas as pl
from jax.experimental.pallas import tpu as pltpu
from jax.experimental.pallas import tpu_sc as plsc

F32 = jnp.float32
I32 = jnp.int32

B = 1024
D = 256
HOP = 10
N1 = B * HOP
NC, NS = 2, 16
NW = NC * NS
IDS_W = B // NW
S1_W = IDS_W * HOP
S2_W = S1_W * HOP
CH = 80
NCH = S2_W // CH
NXCH = S1_W // CH
HG = S1_W // 2
ACC = NS * HG
INV_HOP = 0.1


def _proj_body(x_ref, w_ref, b_ref, o_ref):
    y = jnp.dot(x_ref[...], w_ref[...], preferred_element_type=F32)
    o_ref[...] = 0.5 * jnp.tanh(0.5 * (y + b_ref[...])) + 0.5


def _project(feat, w, b):
    n = feat.shape[0]
    blk = 1000
    return pl.pallas_call(
        _proj_body,
        grid=(n // blk,),
        in_specs=[
            pl.BlockSpec((blk, D), lambda i: (i, 0)),
            pl.BlockSpec((D, D), lambda i: (0, 0)),
            pl.BlockSpec((1, D), lambda i: (0, 0)),
        ],
        out_specs=pl.BlockSpec((blk, D), lambda i: (i, 0)),
        out_shape=jax.ShapeDtypeStruct((n, D), F32),
    )(feat, w, b.reshape(1, D))


def _pos16(off):
    return lax.iota(I32, 16) + off


def _div10(pos):
    return (pos.astype(F32) * F32(0.1)).astype(I32)


def _sc_body(uid, iid, up, ip, u2i, i2u, x0_o, x1_o, m2_o,
             ids_v, s1_v, s2_v, idx_v, rows2_v, acc_v,
             sem_a, sem_b, sem_i, sem_o):
    c = lax.axis_index("c")
    s = lax.axis_index("s")
    wid = c * NS + s
    sems = (sem_a, sem_b)

    sides = ((uid, up, ip, u2i, i2u), (iid, ip, up, i2u, u2i))
    for side, (seed, projA, projB, adjA, adjB) in enumerate(sides):
        pltpu.sync_copy(seed.at[pl.ds(wid * IDS_W, IDS_W)], ids_v)

        for i in range(S1_W // 16):
            pos = _pos16(i * 16)
            r = _div10(pos)
            rb = min((i * 16) // HOP, IDS_W - 16)
            v = ids_v[pl.ds(rb, 16)][r - rb]
            idx_v[pl.ds(i * 16, 16)] = v * 16 + (pos - r * HOP)

        def _g1s(i, _):
            pltpu.async_copy(adjA.at[idx_v.at[pl.ds(i * CH, CH)]],
                             s1_v.at[pl.ds(i * CH, CH)], sem_i)
            return 0
        lax.fori_loop(0, NXCH, _g1s, 0)

        def _g1w(i, _):
            pltpu.make_async_copy(adjA.at[idx_v.at[pl.ds(i * CH, CH)]],
                                  s1_v.at[pl.ds(i * CH, CH)], sem_i).wait()
            return 0
        lax.fori_loop(0, NXCH, _g1w, 0)

        for i in range(S2_W // 16):
            pos = _pos16(i * 16)
            r = _div10(pos)
            rb = min((i * 16) // HOP, S1_W - 16)
            v = s1_v[pl.ds(rb, 16)][r - rb]
            idx_v[pl.ds(i * 16, 16)] = v * 16 + (pos - r * HOP)

        def _g2s(i, _):
            pltpu.async_copy(adjB.at[idx_v.at[pl.ds(i * 128, 128)]],
                             s2_v.at[pl.ds(i * 128, 128)], sem_i)
            return 0
        lax.fori_loop(0, S2_W // 128, _g2s, 0)

        def _g2w(i, _):
            pltpu.make_async_copy(adjB.at[idx_v.at[pl.ds(i * 128, 128)]],
                                  s2_v.at[pl.ds(i * 128, 128)], sem_i).wait()
            return 0
        lax.fori_loop(0, S2_W // 128, _g2w, 0)

        pltpu.sync_copy(projA.at[ids_v], rows2_v.at[pl.ds(0, IDS_W)])
        pltpu.sync_copy(rows2_v.at[pl.ds(0, IDS_W)],
                        x0_o.at[side].at[pl.ds(wid * IDS_W, IDS_W)])

        def _x1out(i):
            h = (i % 2) * CH
            return pltpu.make_async_copy(
                rows2_v.at[pl.ds(h, CH)],
                x1_o.at[side].at[pl.ds(wid * S1_W + i * CH, CH)], sem_o)
        for i in range(NXCH):
            h = (i % 2) * CH
            if i >= 2:
                _x1out(i - 2).wait()
            pltpu.sync_copy(projB.at[s1_v.at[pl.ds(i * CH, CH)]],
                            rows2_v.at[pl.ds(h, CH)])
            pltpu.async_copy(rows2_v.at[pl.ds(h, CH)],
                             x1_o.at[side].at[pl.ds(wid * S1_W + i * CH, CH)],
                             sem_o)
        for i in range(NXCH - 2, NXCH):
            _x1out(i).wait()

        for p in range(2):
            half0 = p * (NCH // 2)

            def _gather(j, b):
                return pltpu.async_copy(
                    projA.at[s2_v.at[pl.ds(j * CH, CH)]],
                    rows2_v.at[pl.ds(b * CH, CH)], sems[b])

            for b in range(2):
                _gather(half0 + b, b)

            def _ring(gg, _, half0=half0):
                for b in range(2):
                    j = half0 + 2 * gg + b
                    pltpu.make_async_copy(
                        projA.at[s2_v.at[pl.ds(j * CH, CH)]],
                        rows2_v.at[pl.ds(b * CH, CH)], sems[b]).wait()

                    def _grp(g, _, b=b, gg=gg):
                        row0 = b * CH + g * HOP

                        def _col(cq, _):
                            cs = pl.ds(cq * 16, 16)
                            x = rows2_v[row0, cs]
                            for t in range(1, HOP):
                                x = x + rows2_v[row0 + t, cs]
                            acc_v[16 * gg + 8 * b + g, cs] = x
                            return 0
                        return lax.fori_loop(0, D // 16, _col, 0)
                    lax.fori_loop(0, CH // HOP, _grp, 0)

                    @pl.when(gg < NCH // 4 - 1)
                    def _():
                        _gather(j + 2, b)
                return 0
            lax.fori_loop(0, NCH // 4, _ring, 0)

            pltpu.sync_copy(
                acc_v, m2_o.at[side].at[pl.ds(wid * S1_W + p * HG, HG)])


def _sc_stage(user_id, item_id, up, ip, u2i, i2u):
    mesh = plsc.VectorSubcoreMesh(core_axis_name="c", subcore_axis_name="s")
    fn = pl.kernel(
        _sc_body,
        out_type=[
            jax.ShapeDtypeStruct((2, B, D), F32),
            jax.ShapeDtypeStruct((2, N1, D), F32),
            jax.ShapeDtypeStruct((2, N1, D), F32),
        ],
        mesh=mesh,
        scratch_types=[
            pltpu.VMEM((IDS_W,), I32),
            pltpu.VMEM((S1_W,), I32),
            pltpu.VMEM((S2_W,), I32),
            pltpu.VMEM((S2_W,), I32),
            pltpu.VMEM((2 * CH, D), F32),
            pltpu.VMEM((HG, D), F32),
            pltpu.SemaphoreType.DMA,
            pltpu.SemaphoreType.DMA,
            pltpu.SemaphoreType.DMA,
            pltpu.SemaphoreType.DMA,
        ],
    )
    return fn(user_id, item_id, up, ip, u2i, i2u)


def _l0_body(x1_ref, m2_ref, ws_ref, wa_ref, h1s_ref, xs1_ref):
    x1 = x1_ref[0]
    m2 = m2_ref[0] * INV_HOP
    h1 = jnp.dot(x1, ws_ref[0], preferred_element_type=F32)
    h1 = h1 + jnp.dot(m2, wa_ref[0], preferred_element_type=F32)
    h1 = jnp.maximum(h1, 0.0)
    rows = h1.shape[0] // HOP
    h1s_ref[0] = h1.reshape(rows, HOP, D).sum(axis=1)
    xs1_ref[0] = x1.reshape(rows, HOP, D).sum(axis=1)


def _layer0(x1s, m2s, ws0, wa0):
    blk = 1280
    nblk = N1 // blk
    return pl.pallas_call(
        _l0_body,
        grid=(2, nblk),
        in_specs=[
            pl.BlockSpec((1, blk, D), lambda g, i: (g, i, 0)),
            pl.BlockSpec((1, blk, D), lambda g, i: (g, i, 0)),
            pl.BlockSpec((1, D, D), lambda g, i: (g, 0, 0)),
            pl.BlockSpec((1, D, D), lambda g, i: (g, 0, 0)),
        ],
        out_specs=[
            pl.BlockSpec((1, blk // HOP, D), lambda g, i: (g, i, 0)),
            pl.BlockSpec((1, blk // HOP, D), lambda g, i: (g, i, 0)),
        ],
        out_shape=[
            jax.ShapeDtypeStruct((2, B, D), F32),
            jax.ShapeDtypeStruct((2, B, D), F32),
        ],
    )(x1s, m2s, ws0, wa0)


def _head_body(x0_ref, xs1_ref, h1s_ref, ws0_ref, wa0_ref, ws1_ref, wa1_ref,
               wl_ref, bl_ref, o_ref):
    outs = []
    for g in range(2):
        h0 = jnp.dot(x0_ref[g], ws0_ref[g], preferred_element_type=F32)
        h0 = h0 + jnp.dot(xs1_ref[g] * INV_HOP, wa0_ref[g],
                          preferred_element_type=F32)
        h0 = jnp.maximum(h0, 0.0)
        hid = jnp.dot(h0, ws1_ref[g], preferred_element_type=F32)
        hid = hid + jnp.dot(h1s_ref[g] * INV_HOP, wa1_ref[g],
                            preferred_element_type=F32)
        outs.append(hid)
    prod = outs[0] * outs[1]
    o_ref[...] = jnp.sum(prod * wl_ref[...], axis=-1,
                         keepdims=True) + bl_ref[...]


def _head(x0s, xs1, h1s, ws0, wa0, ws1, wa1, wl, bl):
    h1 = ws1.shape[-1]
    return pl.pallas_call(
        _head_body,
        out_shape=jax.ShapeDtypeStruct((B, 1), F32),
    )(x0s, xs1, h1s, ws0, wa0, ws1, wa1, wl.reshape(1, h1), bl.reshape(1, 1))


def kernel(user_id, item_id, user_feat, item_feat, edge_feature, num_sampling,
           user2item, item2user, W_proj_u, b_proj_u, W_proj_i, b_proj_i,
           Wu_self0, Wu_agg0, Wu_self1, Wu_agg1, Wi_self0, Wi_agg0, Wi_self1,
           Wi_agg1, W_lin, b_lin):
    up = _project(user_feat, W_proj_u, b_proj_u)
    ip = _project(item_feat, W_proj_i, b_proj_i)

    x0s, x1s, m2s = _sc_stage(user_id.astype(I32), item_id.astype(I32),
                              up, ip, user2item.astype(I32).reshape(-1),
                              item2user.astype(I32).reshape(-1))

    ws0 = jnp.stack([Wu_self0, Wi_self0])
    wa0 = jnp.stack([Wu_agg0, Wi_agg0])
    ws1 = jnp.stack([Wu_self1, Wi_self1])
    wa1 = jnp.stack([Wu_agg1, Wi_agg1])

    h1s, xs1 = _layer0(x1s, m2s, ws0, wa0)
    pred = _head(x0s, xs1, h1s, ws0, wa0, ws1, wa1, W_lin, b_lin)
    return (pred, num_sampling)
